# Initial kernel scaffold; baseline (speedup 1.0000x reference)
import jax
import jax.numpy as jnp
from jax.experimental import pallas as pl


def kernel(x, position_ids, cos_sin_cache):
    raise NotImplementedError("write your pallas kernel here")


# speedup vs baseline: 1.6940x; 1.6940x over previous
import functools

import jax
import jax.numpy as jnp
from jax import lax
from jax.experimental import pallas as pl
from jax.experimental.pallas import tpu as pltpu
from jax.experimental.pallas import tpu_sc as plsc

_info = plsc.get_sparse_core_info()
_NC, _NS = _info.num_cores, _info.num_subcores
_NW = _NC * _NS

_CHUNK = 128


def _make_gather(total, d2, n_chunks):
    d_half = d2 // 2
    mesh = plsc.VectorSubcoreMesh(core_axis_name="c", subcore_axis_name="s")

    @functools.partial(
        pl.kernel,
        out_type=(
            jax.ShapeDtypeStruct((total, d_half), jnp.float32),
            jax.ShapeDtypeStruct((total, d_half), jnp.float32),
        ),
        mesh=mesh,
        scratch_types=[
            pltpu.VMEM((n_chunks, _CHUNK), jnp.int32),
            pltpu.VMEM((_CHUNK, d2), jnp.float32),
            pltpu.SemaphoreType.DMA,
        ],
    )
    def gather_kernel(table_hbm, idx_hbm, cos_hbm, sin_hbm, idx_v, rows_v, sem):
        wid = lax.axis_index("s") * _NC + lax.axis_index("c")
        pltpu.sync_copy(idx_hbm.at[wid], idx_v)
        for c in range(n_chunks):
            base = wid * (n_chunks * _CHUNK) + c * _CHUNK
            pltpu.async_copy(table_hbm.at[idx_v.at[c]], rows_v, sem).wait()
            pltpu.sync_copy(rows_v.at[:, pl.ds(0, d_half)],
                            cos_hbm.at[pl.ds(base, _CHUNK)])
            pltpu.sync_copy(rows_v.at[:, pl.ds(d_half, d_half)],
                            sin_hbm.at[pl.ds(base, _CHUNK)])

    return gather_kernel


def kernel(x, position_ids, cos_sin_cache):
    if position_ids.ndim == 3:
        position_ids = position_ids[0]
    b, s = position_ids.shape
    total = b * s
    d2 = cos_sin_cache.shape[-1]
    n_chunks = total // (_NW * _CHUNK)
    idx = position_ids.reshape(_NW, n_chunks, _CHUNK)
    cos_flat, sin_flat = _make_gather(total, d2, n_chunks)(cos_sin_cache, idx)
    d_half = d2 // 2
    return (cos_flat.reshape(b, s, d_half), sin_flat.reshape(b, s, d_half))


# speedup vs baseline: 1.9096x; 1.1273x over previous
import functools

import jax
import jax.numpy as jnp
from jax import lax
from jax.experimental import pallas as pl
from jax.experimental.pallas import tpu as pltpu
from jax.experimental.pallas import tpu_sc as plsc

_info = plsc.get_sparse_core_info()
_NC, _NS = _info.num_cores, _info.num_subcores
_NW = _NC * _NS

_CHUNK = 128


_NBUF = 3


def _make_gather(total, d2, n_chunks):
    d_half = d2 // 2
    mesh = plsc.VectorSubcoreMesh(core_axis_name="c", subcore_axis_name="s")

    @functools.partial(
        pl.kernel,
        out_type=(
            jax.ShapeDtypeStruct((total, d_half), jnp.float32),
            jax.ShapeDtypeStruct((total, d_half), jnp.float32),
        ),
        mesh=mesh,
        scratch_types=[
            pltpu.VMEM((n_chunks, _CHUNK), jnp.int32),
            [pltpu.VMEM((_CHUNK, d2), jnp.float32) for _ in range(_NBUF)],
            [pltpu.SemaphoreType.DMA for _ in range(_NBUF)],
            [pltpu.SemaphoreType.DMA for _ in range(_NBUF)],
        ],
    )
    def gather_kernel(table_hbm, idx_hbm, cos_hbm, sin_hbm,
                      idx_v, rows, sem_g, sem_s):
        wid = lax.axis_index("s") * _NC + lax.axis_index("c")
        pltpu.sync_copy(idx_hbm.at[wid], idx_v)

        def start_gather(c):
            b = c % _NBUF
            return pltpu.async_copy(table_hbm.at[idx_v.at[c]], rows[b], sem_g[b])

        gather = [None] * _NBUF
        scatter = [None] * _NBUF
        for c in range(min(_NBUF, n_chunks)):
            gather[c % _NBUF] = start_gather(c)
        for c in range(n_chunks):
            b = c % _NBUF
            base = wid * (n_chunks * _CHUNK) + c * _CHUNK
            gather[b].wait()
            scatter[b] = (
                pltpu.async_copy(rows[b].at[:, pl.ds(0, d_half)],
                                 cos_hbm.at[pl.ds(base, _CHUNK)], sem_s[b]),
                pltpu.async_copy(rows[b].at[:, pl.ds(d_half, d_half)],
                                 sin_hbm.at[pl.ds(base, _CHUNK)], sem_s[b]),
            )
            nxt = c + _NBUF
            if nxt < n_chunks:
                nb = nxt % _NBUF
                s = scatter[nb]
                if s is not None:
                    s[0].wait()
                    s[1].wait()
                gather[nb] = start_gather(nxt)
        for b in range(min(_NBUF, n_chunks)):
            s = scatter[b]
            if s is not None:
                s[0].wait()
                s[1].wait()

    return gather_kernel


def kernel(x, position_ids, cos_sin_cache):
    if position_ids.ndim == 3:
        position_ids = position_ids[0]
    b, s = position_ids.shape
    total = b * s
    d2 = cos_sin_cache.shape[-1]
    n_chunks = total // (_NW * _CHUNK)
    idx = position_ids.reshape(_NW, n_chunks, _CHUNK)
    cos_flat, sin_flat = _make_gather(total, d2, n_chunks)(cos_sin_cache, idx)
    d_half = d2 // 2
    return (cos_flat.reshape(b, s, d_half), sin_flat.reshape(b, s, d_half))
